# baseline (speedup 1.0000x reference)
import functools

import jax
import jax.numpy as jnp
from jax import lax
from jax.experimental import pallas as pl
from jax.experimental.pallas import tpu as pltpu
from jax.experimental.pallas import tpu_sc as plsc

USER = 60000
ITEM = 40000
LATDIM = 32
HYPERNUM = 128
N = USER + ITEM
E = 1600000
LEAKY = 0.5

NC = 2
NS = 16
DH = LATDIM // 2
EPT = E // NS
CH = 800
G = CH
NB = EPT // G
RPC = EPT // CH
RPT = N // NS

_f32 = jnp.float32


def _leaky(x):
    return jnp.where(x >= 0, x, LEAKY * x)


def _sc_body(table, rec2, out1, out2, acc, recbuf, rows, rsems, gsems, ssems):
    c = lax.axis_index("c")
    s = lax.axis_index("s")

    def zero_acc():
        def _zfill(i, _):
            rows[0, i, :] = jnp.zeros((16,), _f32)
            return _
        lax.fori_loop(0, G, _zfill, None)
        nfull, rem = divmod(RPT, G)
        for k in range(nfull):
            pltpu.sync_copy(rows.at[0], acc.at[pl.ds(s * RPT + k * G, G)])
        if rem:
            pltpu.sync_copy(rows.at[0, pl.ds(0, rem)],
                            acc.at[pl.ds(s * RPT + nfull * G, rem)])

    def layer(tbl_ref, out_ref):
        def rec_dma(b, p2):
            return pltpu.make_async_copy(
                rec2.at[c, s * RPC + b], recbuf.at[p2], rsems.at[p2])

        def gath(b, p2):
            return pltpu.make_async_copy(
                tbl_ref.at[recbuf.at[p2, 0]], rows.at[p2], gsems.at[p2])

        def scat(b, p2):
            return pltpu.make_async_copy(
                rows.at[p2], acc.at[recbuf.at[p2, 1]], ssems.at[p2])

        rec_dma(0, 0).start()
        rec_dma(0, 0).wait()
        gath(0, 0).start()

        def block(b, carry):
            p2 = lax.rem(b, 2)
            p2n = lax.rem(b + 1, 2)

            @pl.when(b >= 1)
            def _():
                scat(b - 1, p2n).wait()

            @pl.when(b < NB - 1)
            def _():
                rec_dma(b + 1, p2n).start()

            gath(b, p2).wait()

            for g in range(G // 16):
                v16 = plsc.bitcast(recbuf[p2, 2, pl.ds(g * 16, 16)], _f32)
                for l in range(16):
                    e = g * 16 + l
                    rows[p2, e, :] = rows[p2, e, :] * v16[l]

            @pl.when(b < NB - 1)
            def _():
                rec_dma(b + 1, p2n).wait()
                gath(b + 1, p2n).start()

            pltpu.async_copy(rows.at[p2], acc.at[recbuf.at[p2, 1]],
                             ssems.at[p2], add=True)
            return carry
        lax.fori_loop(0, NB, block, None)
        scat(NB - 1, lax.rem(NB - 1, 2)).wait()
        plsc.subcore_barrier()
        pltpu.sync_copy(acc.at[pl.ds(s * RPT, RPT)],
                        out_ref.at[pl.ds(c * N + s * RPT, RPT)])

    zero_acc()
    plsc.subcore_barrier()
    layer(table, out1)
    plsc.subcore_barrier()
    zero_acc()
    plsc.subcore_barrier()
    layer(out1, out2)


def _sc_spmm2(table, rec2):
    k = pl.kernel(
        _sc_body,
        out_type=[jax.ShapeDtypeStruct((2 * N, DH), _f32),
                  jax.ShapeDtypeStruct((2 * N, DH), _f32)],
        mesh=plsc.VectorSubcoreMesh(core_axis_name="c", subcore_axis_name="s"),
        scratch_types=[
            pltpu.VMEM_SHARED((N, DH), _f32),
            pltpu.VMEM((2, 3, CH), jnp.int32),
            pltpu.VMEM((2, G, DH), _f32),
            pltpu.SemaphoreType.DMA((2,)),
            pltpu.SemaphoreType.DMA((2,)),
            pltpu.SemaphoreType.DMA((2,)),
        ],
        compiler_params=pltpu.CompilerParams(use_tc_tiling_on_sc=False,
                                             needs_layout_passes=False),
    )
    return k(table, rec2)


R = 2000
NBLK = N // R
UB = USER // R

_dot = functools.partial(lax.dot_general, precision=lax.Precision.HIGHEST,
                         preferred_element_type=_f32)


def _stats_body(x_ref, g_ref):
    i = pl.program_id(0)

    @pl.when(i == 0)
    def _():
        g_ref[...] = jnp.zeros_like(g_ref)

    x = x_ref[...]
    g = _dot(x, x, (((0,), (0,)), ((), ())))

    @pl.when(i < UB)
    def _():
        g_ref[0] += g

    @pl.when(i >= UB)
    def _():
        g_ref[1] += g


def _head1_body(g_ref, uh_ref, ih_ref, m_ref):
    for side, hyp_ref in ((0, uh_ref), (1, ih_ref)):
        hyp = hyp_ref[...]
        a = _dot(hyp, g_ref[side], (((0,), (0,)), ((), ())))
        m_ref[side] = _dot(hyp, a, (((1,), (0,)), ((), ())))


def _hpass_body(x_ref, m_ref, h_ref, p_ref):
    i = pl.program_id(0)

    @pl.when(i == 0)
    def _():
        p_ref[...] = jnp.zeros_like(p_ref)

    x = x_ref[...]
    m = jnp.where(i < UB, m_ref[0], m_ref[1])
    h = _leaky(_dot(x, m, (((1,), (0,)), ((), ()))))
    h_ref[...] = h + x
    p = _dot(x, h, (((0,), (0,)), ((), ())))

    @pl.when(i < UB)
    def _():
        p_ref[0] += p

    @pl.when(i >= UB)
    def _():
        p_ref[1] += p


def _head2_body(g_ref, p_ref, uh_ref, ih_ref, q_ref):
    for side, hyp_ref in ((0, uh_ref), (1, ih_ref)):
        hyp = hyp_ref[...]
        g = g_ref[side]
        p = p_ref[side]
        hue = 2.0 * _dot(hyp, p + g, (((0,), (0,)), ((), ())))
        huet = 2.0 * _dot(hyp, p, (((0,), (0,)), ((), ())))
        an = hue / (jnp.sqrt(jnp.sum(hue * hue, axis=1, keepdims=True)) + 1e-8)
        bn = huet / (jnp.sqrt(jnp.sum(huet * huet, axis=1, keepdims=True)) + 1e-8)
        simi = _dot(an, bn, (((1,), (1,)), ((), ())))
        w = _dot(hyp, simi, (((1,), (0,)), ((), ())))
        b = _dot(w, g, (((0,), (0,)), ((), ())))
        q_ref[side] = _dot(w, b, (((1,), (0,)), ((), ())))


def _final_body(x_ref, s1_ref, s2_ref, h_ref, q_ref, out_ref):
    i = pl.program_id(0)
    x = x_ref[...]
    s1 = jnp.concatenate([s1_ref[0], s1_ref[1]], axis=1)
    s2 = jnp.concatenate([s2_ref[0], s2_ref[1]], axis=1)
    q = jnp.where(i < UB, q_ref[0], q_ref[1])
    l = _leaky(_dot(x, q, (((1,), (0,)), ((), ())))) + x
    h = h_ref[...]
    out_ref[0] = x + s1 + s2
    out_ref[1] = s1
    out_ref[2] = s2
    out_ref[3] = h
    out_ref[4] = h
    out_ref[5] = l
    out_ref[6] = l


def _small(shape):
    return pl.BlockSpec(shape, lambda *_: (0,) * len(shape))


def kernel(adj_indices, adj_values, keepRate, uEmbeds, iEmbeds, uHyper, iHyper):
    del keepRate
    embeds = jnp.concatenate([uEmbeds, iEmbeds], axis=0)
    table = jnp.concatenate([embeds[:, :DH], embeds[:, DH:]], axis=0)
    cols = adj_indices[1].astype(jnp.int32).reshape(E // CH, CH)
    dstr = adj_indices[0].astype(jnp.int32).reshape(E // CH, CH)
    vbits = lax.bitcast_convert_type(adj_values.astype(_f32),
                                     jnp.int32).reshape(E // CH, CH)
    rec2 = jnp.stack([
        jnp.stack([cols, dstr, vbits], axis=1),
        jnp.stack([cols + N, dstr, vbits], axis=1),
    ])

    s1h, s2h = _sc_spmm2(table, rec2)
    s1v = s1h.reshape(2, N, DH)
    s2v = s2h.reshape(2, N, DH)

    gmat = pl.pallas_call(
        _stats_body,
        grid=(NBLK,),
        in_specs=[pl.BlockSpec((R, LATDIM), lambda i: (i, 0))],
        out_specs=_small((2, LATDIM, LATDIM)),
        out_shape=jax.ShapeDtypeStruct((2, LATDIM, LATDIM), _f32),
    )(embeds)

    mmat = pl.pallas_call(
        _head1_body,
        in_specs=[_small((2, LATDIM, LATDIM)),
                  _small((LATDIM, HYPERNUM)),
                  _small((LATDIM, HYPERNUM))],
        out_specs=_small((2, LATDIM, LATDIM)),
        out_shape=jax.ShapeDtypeStruct((2, LATDIM, LATDIM), _f32),
    )(gmat, uHyper, iHyper)

    hmat, pmat = pl.pallas_call(
        _hpass_body,
        grid=(NBLK,),
        in_specs=[pl.BlockSpec((R, LATDIM), lambda i: (i, 0)),
                  _small((2, LATDIM, LATDIM))],
        out_specs=[pl.BlockSpec((R, LATDIM), lambda i: (i, 0)),
                   _small((2, LATDIM, LATDIM))],
        out_shape=[jax.ShapeDtypeStruct((N, LATDIM), _f32),
                   jax.ShapeDtypeStruct((2, LATDIM, LATDIM), _f32)],
    )(embeds, mmat)

    qmat = pl.pallas_call(
        _head2_body,
        in_specs=[_small((2, LATDIM, LATDIM)),
                  _small((2, LATDIM, LATDIM)),
                  _small((LATDIM, HYPERNUM)),
                  _small((LATDIM, HYPERNUM))],
        out_specs=_small((2, LATDIM, LATDIM)),
        out_shape=jax.ShapeDtypeStruct((2, LATDIM, LATDIM), _f32),
    )(gmat, pmat, uHyper, iHyper)

    out = pl.pallas_call(
        _final_body,
        grid=(NBLK,),
        in_specs=[pl.BlockSpec((R, LATDIM), lambda i: (i, 0)),
                  pl.BlockSpec((2, R, DH), lambda i: (0, i, 0)),
                  pl.BlockSpec((2, R, DH), lambda i: (0, i, 0)),
                  pl.BlockSpec((R, LATDIM), lambda i: (i, 0)),
                  _small((2, LATDIM, LATDIM))],
        out_specs=pl.BlockSpec((7, R, LATDIM), lambda i: (0, i, 0)),
        out_shape=jax.ShapeDtypeStruct((7, N, LATDIM), _f32),
    )(embeds, s1v, s2v, hmat, qmat)
    return out


# speedup vs baseline: 1.0967x; 1.0967x over previous
import functools

import jax
import jax.numpy as jnp
from jax import lax
from jax.experimental import pallas as pl
from jax.experimental.pallas import tpu as pltpu
from jax.experimental.pallas import tpu_sc as plsc

USER = 60000
ITEM = 40000
LATDIM = 32
HYPERNUM = 128
N = USER + ITEM
E = 1600000
LEAKY = 0.5

NC = 2
NS = 16
DH = LATDIM // 2
EPT = E // NS
CH = 400
G = CH
NB = EPT // G
RPC = EPT // CH
RPT = N // NS

_f32 = jnp.float32


def _leaky(x):
    return jnp.where(x >= 0, x, LEAKY * x)


def _sc_body(table, rec2, out1, out2, acc, recbuf, rows, rsems, gsems, ssems):
    c = lax.axis_index("c")
    s = lax.axis_index("s")

    def zero_acc():
        def _zfill(i, _):
            rows[0, i, :] = jnp.zeros((16,), _f32)
            return _
        lax.fori_loop(0, G, _zfill, None)
        nfull, rem = divmod(RPT, G)
        for k in range(nfull):
            pltpu.sync_copy(rows.at[0], acc.at[pl.ds(s * RPT + k * G, G)])
        if rem:
            pltpu.sync_copy(rows.at[0, pl.ds(0, rem)],
                            acc.at[pl.ds(s * RPT + nfull * G, rem)])

    def layer(tbl_ref, out_ref):
        def rec_dma(b, p3):
            return pltpu.make_async_copy(
                rec2.at[c, s * RPC + b], recbuf.at[p3], rsems.at[p3])

        def gath(b, p3, p2):
            return pltpu.make_async_copy(
                tbl_ref.at[recbuf.at[p3, 0]], rows.at[p2], gsems.at[p2])

        def scat(b, p3, p2):
            return pltpu.make_async_copy(
                rows.at[p2], acc.at[recbuf.at[p3, 1]], ssems.at[p2])

        rec_dma(0, 0).start()
        rec_dma(1, 1).start()
        rec_dma(0, 0).wait()
        gath(0, 0, 0).start()

        def block(b, carry):
            p3 = lax.rem(b, 3)
            p3n = lax.rem(b + 1, 3)
            p3nn = lax.rem(b + 2, 3)
            p2 = lax.rem(b, 2)
            p2n = lax.rem(b + 1, 2)

            @pl.when(b >= 1)
            def _():
                scat(b - 1, p3nn, p2n).wait()

            @pl.when(b < NB - 1)
            def _():
                rec_dma(b + 1, p3n).wait()
                gath(b + 1, p3n, p2n).start()

            @pl.when(b < NB - 2)
            def _():
                rec_dma(b + 2, p3nn).start()

            gath(b, p3, p2).wait()

            for g in range(G // 16):
                v16 = plsc.bitcast(recbuf[p3, 2, pl.ds(g * 16, 16)], _f32)
                for l in range(16):
                    e = g * 16 + l
                    rows[p2, e, :] = rows[p2, e, :] * v16[l]

            pltpu.async_copy(rows.at[p2], acc.at[recbuf.at[p3, 1]],
                             ssems.at[p2], add=True)
            return carry
        lax.fori_loop(0, NB, block, None)
        scat(NB - 1, lax.rem(NB - 1, 3), lax.rem(NB - 1, 2)).wait()
        plsc.subcore_barrier()
        pltpu.sync_copy(acc.at[pl.ds(s * RPT, RPT)],
                        out_ref.at[pl.ds(c * N + s * RPT, RPT)])

    zero_acc()
    plsc.subcore_barrier()
    layer(table, out1)
    plsc.subcore_barrier()
    zero_acc()
    plsc.subcore_barrier()
    layer(out1, out2)


def _sc_spmm2(table, rec2):
    k = pl.kernel(
        _sc_body,
        out_type=[jax.ShapeDtypeStruct((2 * N, DH), _f32),
                  jax.ShapeDtypeStruct((2 * N, DH), _f32)],
        mesh=plsc.VectorSubcoreMesh(core_axis_name="c", subcore_axis_name="s"),
        scratch_types=[
            pltpu.VMEM_SHARED((N, DH), _f32),
            pltpu.VMEM((3, 3, CH), jnp.int32),
            pltpu.VMEM((2, G, DH), _f32),
            pltpu.SemaphoreType.DMA((3,)),
            pltpu.SemaphoreType.DMA((2,)),
            pltpu.SemaphoreType.DMA((2,)),
        ],
        compiler_params=pltpu.CompilerParams(use_tc_tiling_on_sc=False,
                                             needs_layout_passes=False),
    )
    return k(table, rec2)


R = 2000
NBLK = N // R
UB = USER // R

_dot = functools.partial(lax.dot_general, precision=lax.Precision.HIGHEST,
                         preferred_element_type=_f32)


def _stats_body(x_ref, g_ref):
    i = pl.program_id(0)

    @pl.when(i == 0)
    def _():
        g_ref[...] = jnp.zeros_like(g_ref)

    x = x_ref[...]
    g = _dot(x, x, (((0,), (0,)), ((), ())))

    @pl.when(i < UB)
    def _():
        g_ref[0] += g

    @pl.when(i >= UB)
    def _():
        g_ref[1] += g


def _head1_body(g_ref, uh_ref, ih_ref, m_ref):
    for side, hyp_ref in ((0, uh_ref), (1, ih_ref)):
        hyp = hyp_ref[...]
        a = _dot(hyp, g_ref[side], (((0,), (0,)), ((), ())))
        m_ref[side] = _dot(hyp, a, (((1,), (0,)), ((), ())))


def _hpass_body(x_ref, m_ref, h_ref, p_ref):
    i = pl.program_id(0)

    @pl.when(i == 0)
    def _():
        p_ref[...] = jnp.zeros_like(p_ref)

    x = x_ref[...]
    m = jnp.where(i < UB, m_ref[0], m_ref[1])
    h = _leaky(_dot(x, m, (((1,), (0,)), ((), ()))))
    h_ref[...] = h + x
    p = _dot(x, h, (((0,), (0,)), ((), ())))

    @pl.when(i < UB)
    def _():
        p_ref[0] += p

    @pl.when(i >= UB)
    def _():
        p_ref[1] += p


def _head2_body(g_ref, p_ref, uh_ref, ih_ref, q_ref):
    for side, hyp_ref in ((0, uh_ref), (1, ih_ref)):
        hyp = hyp_ref[...]
        g = g_ref[side]
        p = p_ref[side]
        hue = 2.0 * _dot(hyp, p + g, (((0,), (0,)), ((), ())))
        huet = 2.0 * _dot(hyp, p, (((0,), (0,)), ((), ())))
        an = hue / (jnp.sqrt(jnp.sum(hue * hue, axis=1, keepdims=True)) + 1e-8)
        bn = huet / (jnp.sqrt(jnp.sum(huet * huet, axis=1, keepdims=True)) + 1e-8)
        simi = _dot(an, bn, (((1,), (1,)), ((), ())))
        w = _dot(hyp, simi, (((1,), (0,)), ((), ())))
        b = _dot(w, g, (((0,), (0,)), ((), ())))
        q_ref[side] = _dot(w, b, (((1,), (0,)), ((), ())))


def _final_body(x_ref, s1_ref, s2_ref, h_ref, q_ref, out_ref):
    i = pl.program_id(0)
    x = x_ref[...]
    s1 = jnp.concatenate([s1_ref[0], s1_ref[1]], axis=1)
    s2 = jnp.concatenate([s2_ref[0], s2_ref[1]], axis=1)
    q = jnp.where(i < UB, q_ref[0], q_ref[1])
    l = _leaky(_dot(x, q, (((1,), (0,)), ((), ())))) + x
    h = h_ref[...]
    out_ref[0] = x + s1 + s2
    out_ref[1] = s1
    out_ref[2] = s2
    out_ref[3] = h
    out_ref[4] = h
    out_ref[5] = l
    out_ref[6] = l


def _small(shape):
    return pl.BlockSpec(shape, lambda *_: (0,) * len(shape))


def kernel(adj_indices, adj_values, keepRate, uEmbeds, iEmbeds, uHyper, iHyper):
    del keepRate
    embeds = jnp.concatenate([uEmbeds, iEmbeds], axis=0)
    table = jnp.concatenate([embeds[:, :DH], embeds[:, DH:]], axis=0)
    cols = adj_indices[1].astype(jnp.int32).reshape(E // CH, CH)
    dstr = adj_indices[0].astype(jnp.int32).reshape(E // CH, CH)
    vbits = lax.bitcast_convert_type(adj_values.astype(_f32),
                                     jnp.int32).reshape(E // CH, CH)
    rec2 = jnp.stack([
        jnp.stack([cols, dstr, vbits], axis=1),
        jnp.stack([cols + N, dstr, vbits], axis=1),
    ])

    s1h, s2h = _sc_spmm2(table, rec2)
    s1v = s1h.reshape(2, N, DH)
    s2v = s2h.reshape(2, N, DH)

    gmat = pl.pallas_call(
        _stats_body,
        grid=(NBLK,),
        in_specs=[pl.BlockSpec((R, LATDIM), lambda i: (i, 0))],
        out_specs=_small((2, LATDIM, LATDIM)),
        out_shape=jax.ShapeDtypeStruct((2, LATDIM, LATDIM), _f32),
    )(embeds)

    mmat = pl.pallas_call(
        _head1_body,
        in_specs=[_small((2, LATDIM, LATDIM)),
                  _small((LATDIM, HYPERNUM)),
                  _small((LATDIM, HYPERNUM))],
        out_specs=_small((2, LATDIM, LATDIM)),
        out_shape=jax.ShapeDtypeStruct((2, LATDIM, LATDIM), _f32),
    )(gmat, uHyper, iHyper)

    hmat, pmat = pl.pallas_call(
        _hpass_body,
        grid=(NBLK,),
        in_specs=[pl.BlockSpec((R, LATDIM), lambda i: (i, 0)),
                  _small((2, LATDIM, LATDIM))],
        out_specs=[pl.BlockSpec((R, LATDIM), lambda i: (i, 0)),
                   _small((2, LATDIM, LATDIM))],
        out_shape=[jax.ShapeDtypeStruct((N, LATDIM), _f32),
                   jax.ShapeDtypeStruct((2, LATDIM, LATDIM), _f32)],
    )(embeds, mmat)

    qmat = pl.pallas_call(
        _head2_body,
        in_specs=[_small((2, LATDIM, LATDIM)),
                  _small((2, LATDIM, LATDIM)),
                  _small((LATDIM, HYPERNUM)),
                  _small((LATDIM, HYPERNUM))],
        out_specs=_small((2, LATDIM, LATDIM)),
        out_shape=jax.ShapeDtypeStruct((2, LATDIM, LATDIM), _f32),
    )(gmat, pmat, uHyper, iHyper)

    out = pl.pallas_call(
        _final_body,
        grid=(NBLK,),
        in_specs=[pl.BlockSpec((R, LATDIM), lambda i: (i, 0)),
                  pl.BlockSpec((2, R, DH), lambda i: (0, i, 0)),
                  pl.BlockSpec((2, R, DH), lambda i: (0, i, 0)),
                  pl.BlockSpec((R, LATDIM), lambda i: (i, 0)),
                  _small((2, LATDIM, LATDIM))],
        out_specs=pl.BlockSpec((7, R, LATDIM), lambda i: (0, i, 0)),
        out_shape=jax.ShapeDtypeStruct((7, N, LATDIM), _f32),
    )(embeds, s1v, s2v, hmat, qmat)
    return out
